# Initial kernel scaffold; baseline (speedup 1.0000x reference)
import jax
import jax.numpy as jnp
from jax.experimental import pallas as pl


def kernel(indices, weights, mask, table, W1, b1, W2, b2, W3, b3):
    raise NotImplementedError("write your pallas kernel here")


# speedup vs baseline: 8.9300x; 8.9300x over previous
import functools

import jax
import jax.numpy as jnp
import numpy as np
from jax import lax
from jax.experimental import pallas as pl
from jax.experimental.pallas import tpu as pltpu
from jax.experimental.pallas import tpu_sc as plsc

B, L = 16384, 200
TOKEN_DIM = 32
H1, H2, EMBED_DIM = 256, 128, 64

NC, NS = 2, 16
NW = NC * NS
BPW = B // NW
R = 8
CH = R * L
NCHUNK = BPW // R
GS = 128
NG = -(-CH // GS)
CHP = NG * GS

LANES = 16

_PERM = np.concatenate([np.arange(0, TOKEN_DIM, 2), np.arange(1, TOKEN_DIM, 2)])


def _sc_pool(idx_flat, w_flat, m_flat, table_bf):
  mesh = plsc.VectorSubcoreMesh(core_axis_name="c", subcore_axis_name="s")

  @functools.partial(
      pl.kernel,
      out_type=jax.ShapeDtypeStruct((B, TOKEN_DIM), jnp.float32),
      mesh=mesh,
      compiler_params=pltpu.CompilerParams(use_tc_tiling_on_sc=False),
      scratch_types=dict(
          idx_v=pltpu.VMEM((2, CHP), jnp.int32),
          w_v=pltpu.VMEM((2, CHP), jnp.float32),
          m_v=pltpu.VMEM((2, CHP), jnp.float32),
          wc_v=pltpu.VMEM((2, CHP), jnp.float32),
          emb_v=pltpu.VMEM((2, CHP, TOKEN_DIM // 2), jnp.int32),
          out_v=pltpu.VMEM((2, R, TOKEN_DIM), jnp.float32),
          semin0=pltpu.SemaphoreType.DMA,
          semin1=pltpu.SemaphoreType.DMA,
          semg0=pltpu.SemaphoreType.DMA,
          semg1=pltpu.SemaphoreType.DMA,
          semo0=pltpu.SemaphoreType.DMA,
          semo1=pltpu.SemaphoreType.DMA,
      ),
  )
  def pool(idx_hbm, w_hbm, m_hbm, tab_hbm, pooled_hbm, *, idx_v, w_v, m_v,
           wc_v, emb_v, out_v, semin0, semin1, semg0, semg1, semo0, semo1):
    wid = lax.axis_index("s") * NC + lax.axis_index("c")
    base = wid * (BPW * L)
    semin = (semin0, semin1)
    semg = (semg0, semg1)
    semo = (semo0, semo1)

    def in_copies(c, p, sem):
      off = base + c * CH
      return (
          pltpu.make_async_copy(idx_hbm.at[pl.ds(off, CHP)], idx_v.at[p], sem),
          pltpu.make_async_copy(w_hbm.at[pl.ds(off, CHP)], w_v.at[p], sem),
          pltpu.make_async_copy(m_hbm.at[pl.ds(off, CHP)], m_v.at[p], sem),
      )

    def start_in(c, p):
      for d in in_copies(c, p, semin[p]):
        d.start()

    def wait_in(c, p):
      for d in in_copies(c, p, semin[p]):
        d.wait()

    def fold_mask(p):
      @pl.loop(0, CH // LANES)
      def _(k):
        s = pl.ds(k * LANES, LANES)
        wc_v[p, s] = w_v[p, s] * m_v[p, s]

    def gather_copies(p, sem):
      ds = []
      for j in range(NG):
        s = pl.ds(j * GS, GS)
        ds.append(pltpu.make_async_copy(
            tab_hbm.at[idx_v.at[p, s]], emb_v.at[p, s], sem))
      return ds

    def start_gathers(p):
      for d in gather_copies(p, semg[p]):
        d.start()

    def wait_gathers(p):
      for d in gather_copies(p, semg[p]):
        d.wait()

    def out_copy(c, p):
      row0 = wid * BPW + c * R
      return pltpu.make_async_copy(
          out_v.at[p], pooled_hbm.at[pl.ds(row0, R)], semo[p])

    def compute(c, p):
      zero = jnp.zeros((LANES,), jnp.float32)

      hi_mask = jnp.full((LANES,), -65536, jnp.int32)

      def accum(a0, a1, lb, w16, j):
        e = emb_v[p, lb + j, :]
        e0 = lax.bitcast_convert_type(e << 16, jnp.float32)
        e1 = lax.bitcast_convert_type(e & hi_mask, jnp.float32)
        wl = w16[j]
        return a0 + wl * e0, a1 + wl * e1

      def window(a0, a1, lb, lanes):
        w16 = wc_v[p, pl.ds(lb, LANES)]
        for j in lanes:
          a0, a1 = accum(a0, a1, lb, w16, j)
        return a0, a1

      @pl.loop(0, R // 2)
      def _(rr):
        base = rr * (2 * L)

        def body_a(k, carry):
          return window(*carry, base + k * LANES, range(LANES))

        def body_b(k, carry):
          return window(*carry, base + 208 + k * LANES, range(LANES))

        a0, a1 = lax.fori_loop(0, 12, body_a, (zero, zero))
        b0, b1 = lax.fori_loop(0, 12, body_b, (zero, zero))
        w16 = wc_v[p, pl.ds(base + 192, LANES)]
        for j in range(8):
          a0, a1 = accum(a0, a1, base + 192, w16, j)
        for j in range(8, LANES):
          b0, b1 = accum(b0, b1, base + 192, w16, j)
        out_v[p, 2 * rr, 0:LANES] = a0
        out_v[p, 2 * rr, LANES:TOKEN_DIM] = a1
        out_v[p, 2 * rr + 1, 0:LANES] = b0
        out_v[p, 2 * rr + 1, LANES:TOKEN_DIM] = b1

    def stage_a(c, p):
      wait_in(c, p)
      fold_mask(p)
      start_gathers(p)

    def pipe_step(c, p, do_launch=True):
      q = 1 - p
      if do_launch:
        stage_a(c, p)
      wait_gathers(q)

      def _start_next():
        start_in(c + 1, q)
      if isinstance(c, int):
        if c + 1 < NCHUNK:
          _start_next()
      else:
        pl.when(c + 1 < NCHUNK)(_start_next)

      def _wait_prev_out():
        out_copy(c - 3, q).wait()
      if isinstance(c, int):
        if c - 3 >= 0:
          _wait_prev_out()
      else:
        pl.when(c - 3 >= 0)(_wait_prev_out)

      compute(c - 1, q)
      out_copy(c - 1, q).start()

    start_in(0, 0)
    stage_a(0, 0)
    start_in(1, 1)

    @pl.loop(0, (NCHUNK - 2) // 2)
    def _(k):
      c = 2 * k + 1
      pipe_step(c, 1)
      pipe_step(c + 1, 0)

    pipe_step(NCHUNK - 1, 1)
    pipe_step(NCHUNK, 0, do_launch=False)

    out_copy(NCHUNK - 2, 0).wait()
    out_copy(NCHUNK - 1, 1).wait()

  return pool(idx_flat, w_flat, m_flat, table_bf)


def _mlp_body(x_ref, w1_ref, b1_ref, w2_ref, b2_ref, w3_ref, b3_ref, o_ref):
  x = x_ref[...]
  h = jnp.dot(x, w1_ref[...], preferred_element_type=jnp.float32) + b1_ref[...]
  h = jnp.maximum(h, 0.0)
  h = jnp.dot(h, w2_ref[...], preferred_element_type=jnp.float32) + b2_ref[...]
  h = jnp.maximum(h, 0.0)
  out = jnp.dot(h, w3_ref[...], preferred_element_type=jnp.float32) + b3_ref[...]
  norm = jnp.sqrt(jnp.sum(out * out, axis=-1, keepdims=True))
  o_ref[...] = out / jnp.maximum(norm, 1e-12)


def _mlp(pooled, w1, b1, w2, b2, w3, b3):
  blk = 2048
  grid = (B // blk,)
  return pl.pallas_call(
      _mlp_body,
      grid=grid,
      in_specs=[
          pl.BlockSpec((blk, TOKEN_DIM), lambda i: (i, 0)),
          pl.BlockSpec((TOKEN_DIM, H1), lambda i: (0, 0)),
          pl.BlockSpec((1, H1), lambda i: (0, 0)),
          pl.BlockSpec((H1, H2), lambda i: (0, 0)),
          pl.BlockSpec((1, H2), lambda i: (0, 0)),
          pl.BlockSpec((H2, EMBED_DIM), lambda i: (0, 0)),
          pl.BlockSpec((1, EMBED_DIM), lambda i: (0, 0)),
      ],
      out_specs=pl.BlockSpec((blk, EMBED_DIM), lambda i: (i, 0)),
      out_shape=jax.ShapeDtypeStruct((B, EMBED_DIM), jnp.float32),
  )(pooled, w1, b1.reshape(1, H1), w2, b2.reshape(1, H2), w3,
    b3.reshape(1, EMBED_DIM))


def kernel(indices, weights, mask, table, W1, b1, W2, b2, W3, b3):
  idx_flat = jnp.concatenate([
      indices.astype(jnp.int32).reshape(B * L),
      jnp.zeros((CHP - CH,), jnp.int32),
  ])
  pad = jnp.zeros((CHP - CH,), jnp.float32)
  w_flat = jnp.concatenate([weights.reshape(B * L), pad])
  m_flat = jnp.concatenate([mask.astype(jnp.float32).reshape(B * L), pad])
  table_bf = table.astype(jnp.bfloat16)
  table_i32 = lax.bitcast_convert_type(
      table_bf.reshape(-1, TOKEN_DIM // 2, 2), jnp.int32)
  pooled = _sc_pool(idx_flat, w_flat, m_flat, table_i32)
  w1p = W1[_PERM, :]
  return _mlp(pooled, w1p, b1, W2, b2, W3, b3)


# speedup vs baseline: 15.4065x; 1.7252x over previous
import functools

import jax
import jax.numpy as jnp
import numpy as np
from jax import lax
from jax.experimental import pallas as pl
from jax.experimental.pallas import tpu as pltpu
from jax.experimental.pallas import tpu_sc as plsc

B, L = 16384, 200
TOKEN_DIM = 32
H1, H2, EMBED_DIM = 256, 128, 64

NC, NS = 2, 16
NW = NC * NS
BPW = B // NW
R = 8
CH = R * L
NCHUNK = BPW // R
GS = 128
NG = -(-CH // GS)
CHP = NG * GS

LANES = 16


def _sc_pool(idx_flat, w_flat, m_flat, table):
  mesh = plsc.VectorSubcoreMesh(core_axis_name="c", subcore_axis_name="s")

  @functools.partial(
      pl.kernel,
      out_type=jax.ShapeDtypeStruct((B, TOKEN_DIM), jnp.float32),
      mesh=mesh,
      compiler_params=pltpu.CompilerParams(use_tc_tiling_on_sc=False),
      scratch_types=dict(
          idx_v=pltpu.VMEM((2, CHP), jnp.int32),
          w_v=pltpu.VMEM((2, CHP), jnp.float32),
          m_v=pltpu.VMEM((2, CHP), jnp.float32),
          wc_v=pltpu.VMEM((2, CHP), jnp.float32),
          emb_v=pltpu.VMEM((2, CHP, TOKEN_DIM), jnp.float32),
          out_v=pltpu.VMEM((2, R, TOKEN_DIM), jnp.float32),
          semin0=pltpu.SemaphoreType.DMA,
          semin1=pltpu.SemaphoreType.DMA,
          semg0=pltpu.SemaphoreType.DMA,
          semg1=pltpu.SemaphoreType.DMA,
          semo0=pltpu.SemaphoreType.DMA,
          semo1=pltpu.SemaphoreType.DMA,
      ),
  )
  def pool(idx_hbm, w_hbm, m_hbm, tab_hbm, pooled_hbm, *, idx_v, w_v, m_v,
           wc_v, emb_v, out_v, semin0, semin1, semg0, semg1, semo0, semo1):
    wid = lax.axis_index("s") * NC + lax.axis_index("c")
    base = wid * (BPW * L)
    semin = (semin0, semin1)
    semg = (semg0, semg1)
    semo = (semo0, semo1)

    def in_copies(c, p, sem):
      off = base + c * CH
      return (
          pltpu.make_async_copy(idx_hbm.at[pl.ds(off, CHP)], idx_v.at[p], sem),
          pltpu.make_async_copy(w_hbm.at[pl.ds(off, CHP)], w_v.at[p], sem),
          pltpu.make_async_copy(m_hbm.at[pl.ds(off, CHP)], m_v.at[p], sem),
      )

    def start_in(c, p):
      for d in in_copies(c, p, semin[p]):
        d.start()

    def wait_in(c, p):
      for d in in_copies(c, p, semin[p]):
        d.wait()

    def fold_mask(p):
      @pl.loop(0, CH // LANES)
      def _(k):
        s = pl.ds(k * LANES, LANES)
        wc_v[p, s] = w_v[p, s] * m_v[p, s]

    def gather_copies(p, sem):
      ds = []
      for j in range(NG):
        s = pl.ds(j * GS, GS)
        ds.append(pltpu.make_async_copy(
            tab_hbm.at[idx_v.at[p, s]], emb_v.at[p, s], sem))
      return ds

    def start_gathers(p):
      for d in gather_copies(p, semg[p]):
        d.start()

    def wait_gathers(p):
      for d in gather_copies(p, semg[p]):
        d.wait()

    def out_copy(c, p):
      row0 = wid * BPW + c * R
      return pltpu.make_async_copy(
          out_v.at[p], pooled_hbm.at[pl.ds(row0, R)], semo[p])

    def compute(c, p):
      zero = jnp.zeros((LANES,), jnp.float32)

      def accum(a0, a1, lb, w16, j):
        e0 = emb_v[p, lb + j, 0:LANES]
        e1 = emb_v[p, lb + j, LANES:TOKEN_DIM]
        wl = w16[j]
        return a0 + wl * e0, a1 + wl * e1

      def window(a0, a1, lb, lanes):
        w16 = wc_v[p, pl.ds(lb, LANES)]
        for j in lanes:
          a0, a1 = accum(a0, a1, lb, w16, j)
        return a0, a1

      @pl.loop(0, R // 2)
      def _(rr):
        base = rr * (2 * L)

        def body_a(k, carry):
          return window(*carry, base + k * LANES, range(LANES))

        def body_b(k, carry):
          return window(*carry, base + 208 + k * LANES, range(LANES))

        a0, a1 = lax.fori_loop(0, 12, body_a, (zero, zero))
        b0, b1 = lax.fori_loop(0, 12, body_b, (zero, zero))
        w16 = wc_v[p, pl.ds(base + 192, LANES)]
        for j in range(8):
          a0, a1 = accum(a0, a1, base + 192, w16, j)
        for j in range(8, LANES):
          b0, b1 = accum(b0, b1, base + 192, w16, j)
        out_v[p, 2 * rr, 0:LANES] = a0
        out_v[p, 2 * rr, LANES:TOKEN_DIM] = a1
        out_v[p, 2 * rr + 1, 0:LANES] = b0
        out_v[p, 2 * rr + 1, LANES:TOKEN_DIM] = b1

    def stage_a(c, p):
      wait_in(c, p)
      fold_mask(p)
      start_gathers(p)

    def pipe_step(c, p, do_launch=True):
      q = 1 - p
      if do_launch:
        stage_a(c, p)
      wait_gathers(q)

      def _start_next():
        start_in(c + 1, q)
      if isinstance(c, int):
        if c + 1 < NCHUNK:
          _start_next()
      else:
        pl.when(c + 1 < NCHUNK)(_start_next)

      def _wait_prev_out():
        out_copy(c - 3, q).wait()
      if isinstance(c, int):
        if c - 3 >= 0:
          _wait_prev_out()
      else:
        pl.when(c - 3 >= 0)(_wait_prev_out)

      compute(c - 1, q)
      out_copy(c - 1, q).start()

    start_in(0, 0)
    stage_a(0, 0)
    start_in(1, 1)

    @pl.loop(0, (NCHUNK - 2) // 2)
    def _(k):
      c = 2 * k + 1
      pipe_step(c, 1)
      pipe_step(c + 1, 0)

    pipe_step(NCHUNK - 1, 1)
    pipe_step(NCHUNK, 0, do_launch=False)

    out_copy(NCHUNK - 2, 0).wait()
    out_copy(NCHUNK - 1, 1).wait()

  return pool(idx_flat, w_flat, m_flat, table)


def _mlp_body(x_ref, w1_ref, b1_ref, w2_ref, b2_ref, w3_ref, b3_ref, o_ref):
  x = x_ref[...]
  h = jnp.dot(x, w1_ref[...], preferred_element_type=jnp.float32) + b1_ref[...]
  h = jnp.maximum(h, 0.0)
  h = jnp.dot(h, w2_ref[...], preferred_element_type=jnp.float32) + b2_ref[...]
  h = jnp.maximum(h, 0.0)
  out = jnp.dot(h, w3_ref[...], preferred_element_type=jnp.float32) + b3_ref[...]
  norm = jnp.sqrt(jnp.sum(out * out, axis=-1, keepdims=True))
  o_ref[...] = out / jnp.maximum(norm, 1e-12)


def _mlp(pooled, w1, b1, w2, b2, w3, b3):
  blk = 2048
  grid = (B // blk,)
  return pl.pallas_call(
      _mlp_body,
      grid=grid,
      in_specs=[
          pl.BlockSpec((blk, TOKEN_DIM), lambda i: (i, 0)),
          pl.BlockSpec((TOKEN_DIM, H1), lambda i: (0, 0)),
          pl.BlockSpec((1, H1), lambda i: (0, 0)),
          pl.BlockSpec((H1, H2), lambda i: (0, 0)),
          pl.BlockSpec((1, H2), lambda i: (0, 0)),
          pl.BlockSpec((H2, EMBED_DIM), lambda i: (0, 0)),
          pl.BlockSpec((1, EMBED_DIM), lambda i: (0, 0)),
      ],
      out_specs=pl.BlockSpec((blk, EMBED_DIM), lambda i: (i, 0)),
      out_shape=jax.ShapeDtypeStruct((B, EMBED_DIM), jnp.float32),
  )(pooled, w1, b1.reshape(1, H1), w2, b2.reshape(1, H2), w3,
    b3.reshape(1, EMBED_DIM))


def kernel(indices, weights, mask, table, W1, b1, W2, b2, W3, b3):
  idx_flat = jnp.concatenate([
      indices.astype(jnp.int32).reshape(B * L),
      jnp.zeros((CHP - CH,), jnp.int32),
  ])
  pad = jnp.zeros((CHP - CH,), jnp.float32)
  w_flat = jnp.concatenate([weights.reshape(B * L), pad])
  m_flat = jnp.concatenate([mask.astype(jnp.float32).reshape(B * L), pad])
  pooled = _sc_pool(idx_flat, w_flat, m_flat, table)
  return _mlp(pooled, W1, b1, W2, b2, W3, b3)


# speedup vs baseline: 15.5619x; 1.0101x over previous
import functools

import jax
import jax.numpy as jnp
from jax import lax
from jax.experimental import pallas as pl
from jax.experimental.pallas import tpu as pltpu
from jax.experimental.pallas import tpu_sc as plsc

B, L = 16384, 200
TOKEN_DIM = 32
H1, H2, EMBED_DIM = 256, 128, 64

NC, NS = 2, 16
NW = NC * NS
BPW = B // NW
R = 8
CH = R * L
NCHUNK = BPW // R
GS = 128
NG = -(-CH // GS)
CHP = NG * GS

LANES = 16


def _sc_pool(idx_flat, w_flat, table):
  mesh = plsc.VectorSubcoreMesh(core_axis_name="c", subcore_axis_name="s")

  @functools.partial(
      pl.kernel,
      out_type=jax.ShapeDtypeStruct((B, TOKEN_DIM), jnp.float32),
      mesh=mesh,
      compiler_params=pltpu.CompilerParams(use_tc_tiling_on_sc=False),
      scratch_types=dict(
          idx_v=pltpu.VMEM((3, CHP), jnp.int32),
          w_v=pltpu.VMEM((3, CHP), jnp.float32),
          emb_v=pltpu.VMEM((2, CHP, TOKEN_DIM), jnp.float32),
          out_v=pltpu.VMEM((2, R, TOKEN_DIM), jnp.float32),
          semin0=pltpu.SemaphoreType.DMA,
          semin1=pltpu.SemaphoreType.DMA,
          semg0=pltpu.SemaphoreType.DMA,
          semg1=pltpu.SemaphoreType.DMA,
          semo0=pltpu.SemaphoreType.DMA,
          semo1=pltpu.SemaphoreType.DMA,
      ),
  )
  def pool(idx_hbm, w_hbm, tab_hbm, pooled_hbm, *, idx_v, w_v, emb_v, out_v,
           semin0, semin1, semg0, semg1, semo0, semo1):
    wid = lax.axis_index("s") * NC + lax.axis_index("c")
    base = wid * (BPW * L)
    semin = (semin0, semin1)
    semg = (semg0, semg1)
    semo = (semo0, semo1)

    def in_copies(c, sem):
      off = base + c * CH
      slot = c % 3
      return (
          pltpu.make_async_copy(
              idx_hbm.at[pl.ds(off, CHP)], idx_v.at[slot], sem),
          pltpu.make_async_copy(
              w_hbm.at[pl.ds(off, CHP)], w_v.at[slot], sem),
      )

    def start_in(c, p):
      for d in in_copies(c, semin[p]):
        d.start()

    def wait_in(c, p):
      for d in in_copies(c, semin[p]):
        d.wait()

    def gather_copies(c, p, sem):
      slot = c % 3
      ds = []
      for j in range(NG):
        s = pl.ds(j * GS, GS)
        ds.append(pltpu.make_async_copy(
            tab_hbm.at[idx_v.at[slot, s]], emb_v.at[p, s], sem))
      return ds

    def start_gathers(c, p):
      for d in gather_copies(c, p, semg[p]):
        d.start()

    def wait_gathers(c, p):
      for d in gather_copies(c, p, semg[p]):
        d.wait()

    def out_copy(c, p):
      row0 = wid * BPW + c * R
      return pltpu.make_async_copy(
          out_v.at[p], pooled_hbm.at[pl.ds(row0, R)], semo[p])

    def compute(c, p):
      slot = c % 3
      zero = jnp.zeros((LANES,), jnp.float32)

      def accum(a0, a1, lb, w16, j):
        e0 = emb_v[p, lb + j, 0:LANES]
        e1 = emb_v[p, lb + j, LANES:TOKEN_DIM]
        wl = w16[j]
        return a0 + wl * e0, a1 + wl * e1

      def window(a0, a1, lb, lanes):
        w16 = w_v[slot, pl.ds(lb, LANES)]
        for j in lanes:
          a0, a1 = accum(a0, a1, lb, w16, j)
        return a0, a1

      @pl.loop(0, R // 2)
      def _(rr):
        rbase = rr * (2 * L)

        def body_a(k, carry):
          return window(*carry, rbase + k * LANES, range(LANES))

        def body_b(k, carry):
          return window(*carry, rbase + 208 + k * LANES, range(LANES))

        a0, a1 = lax.fori_loop(0, 12, body_a, (zero, zero))
        b0, b1 = lax.fori_loop(0, 12, body_b, (zero, zero))
        w16 = w_v[slot, pl.ds(rbase + 192, LANES)]
        for j in range(8):
          a0, a1 = accum(a0, a1, rbase + 192, w16, j)
        for j in range(8, LANES):
          b0, b1 = accum(b0, b1, rbase + 192, w16, j)
        out_v[p, 2 * rr, 0:LANES] = a0
        out_v[p, 2 * rr, LANES:TOKEN_DIM] = a1
        out_v[p, 2 * rr + 1, 0:LANES] = b0
        out_v[p, 2 * rr + 1, LANES:TOKEN_DIM] = b1

    def pipe_step(c, p, do_launch=True):
      q = 1 - p
      if do_launch:
        wait_in(c, p)
        start_gathers(c, p)

      def _start_next():
        start_in(c + 1, q)
      if isinstance(c, int):
        if c + 1 < NCHUNK:
          _start_next()
      else:
        pl.when(c + 1 < NCHUNK)(_start_next)

      wait_gathers(c - 1, q)

      def _wait_prev_out():
        out_copy(c - 3, q).wait()
      if isinstance(c, int):
        if c - 3 >= 0:
          _wait_prev_out()
      else:
        pl.when(c - 3 >= 0)(_wait_prev_out)

      compute(c - 1, q)
      out_copy(c - 1, q).start()

    start_in(0, 0)
    wait_in(0, 0)
    start_gathers(0, 0)
    start_in(1, 1)

    @pl.loop(0, (NCHUNK - 2) // 2)
    def _(k):
      c = 2 * k + 1
      pipe_step(c, 1)
      pipe_step(c + 1, 0)

    pipe_step(NCHUNK - 1, 1)
    pipe_step(NCHUNK, 0, do_launch=False)

    out_copy(NCHUNK - 2, 0).wait()
    out_copy(NCHUNK - 1, 1).wait()

  return pool(idx_flat, w_flat, table)


def _mlp_body(x_ref, w1_ref, b1_ref, w2_ref, b2_ref, w3_ref, b3_ref, o_ref):
  x = x_ref[...]
  h = jnp.dot(x, w1_ref[...], preferred_element_type=jnp.float32) + b1_ref[...]
  h = jnp.maximum(h, 0.0)
  h = jnp.dot(h, w2_ref[...], preferred_element_type=jnp.float32) + b2_ref[...]
  h = jnp.maximum(h, 0.0)
  out = jnp.dot(h, w3_ref[...], preferred_element_type=jnp.float32) + b3_ref[...]
  norm = jnp.sqrt(jnp.sum(out * out, axis=-1, keepdims=True))
  o_ref[...] = out / jnp.maximum(norm, 1e-12)


def _mlp(pooled, w1, b1, w2, b2, w3, b3):
  blk = 2048
  grid = (B // blk,)
  return pl.pallas_call(
      _mlp_body,
      grid=grid,
      in_specs=[
          pl.BlockSpec((blk, TOKEN_DIM), lambda i: (i, 0)),
          pl.BlockSpec((TOKEN_DIM, H1), lambda i: (0, 0)),
          pl.BlockSpec((1, H1), lambda i: (0, 0)),
          pl.BlockSpec((H1, H2), lambda i: (0, 0)),
          pl.BlockSpec((1, H2), lambda i: (0, 0)),
          pl.BlockSpec((H2, EMBED_DIM), lambda i: (0, 0)),
          pl.BlockSpec((1, EMBED_DIM), lambda i: (0, 0)),
      ],
      out_specs=pl.BlockSpec((blk, EMBED_DIM), lambda i: (i, 0)),
      out_shape=jax.ShapeDtypeStruct((B, EMBED_DIM), jnp.float32),
  )(pooled, w1, b1.reshape(1, H1), w2, b2.reshape(1, H2), w3,
    b3.reshape(1, EMBED_DIM))


def kernel(indices, weights, mask, table, W1, b1, W2, b2, W3, b3):
  pad = jnp.zeros((CHP - CH,), jnp.float32)
  idx_f = lax.bitcast_convert_type(indices.astype(jnp.int32), jnp.float32)
  idx_flat = lax.bitcast_convert_type(
      jnp.concatenate([idx_f.reshape(B * L), pad]), jnp.int32)
  wm = weights * mask.astype(jnp.float32)
  w_flat = jnp.concatenate([wm.reshape(B * L), pad])
  pooled = _sc_pool(idx_flat, w_flat, table)
  return _mlp(pooled, W1, b1, W2, b2, W3, b3)
